# Initial kernel scaffold; baseline (speedup 1.0000x reference)
import jax
import jax.numpy as jnp
from jax.experimental import pallas as pl


def kernel(bert_representation, mtl_input, slice_mask, slice_num, history_attention_input, W):
    raise NotImplementedError("write your pallas kernel here")


# speedup vs baseline: 11.0022x; 11.0022x over previous
import jax
import jax.numpy as jnp
from jax.experimental import pallas as pl

B = 16
T = 11
S = 512
H = 1024


def _fused_step(bert_ref, mtl_ref, hai_ref, w_ref, smf_ref, snz_ref,
                bert_out_ref, mtl_out_ref, probs_out_ref):
    l = jnp.sum(hai_ref[0] * w_ref[...])
    snz = snz_ref[0, 0]
    t = jax.lax.broadcasted_iota(jnp.int32, (1, T), 1).astype(jnp.float32)
    logits = jnp.where(t == float(T - 1), l, 0.0) + snz
    mask = (t >= (float(T) - smf_ref[0, 0, 0])).astype(jnp.float32)
    e = jnp.exp(logits) * mask
    probs = e / jnp.sum(e)
    probs_out_ref[...] = probs.reshape(1, 1, T)
    p = probs[0, T - 1]
    mtl_out_ref[...] = mtl_ref[...] * p
    bert_out_ref[...] = bert_ref[...] * p


def kernel(bert_representation, mtl_input, slice_mask, slice_num,
           history_attention_input, W):
    smf = slice_mask.astype(jnp.float32).reshape(B, 1, 1)
    snz = (jnp.asarray(slice_num) - B).astype(jnp.float32).reshape(1, 1)
    hai3 = history_attention_input.reshape(B, 1, H)
    mtl3 = mtl_input.reshape(B, 1, H)

    out_shapes = (
        jax.ShapeDtypeStruct((B, S, H), jnp.float32),
        jax.ShapeDtypeStruct((B, 1, H), jnp.float32),
        jax.ShapeDtypeStruct((B, 1, T), jnp.float32),
    )
    bert_out, mtl_out, probs_out = pl.pallas_call(
        _fused_step,
        grid=(B,),
        in_specs=[
            pl.BlockSpec((1, S, H), lambda i: (i, 0, 0)),
            pl.BlockSpec((1, 1, H), lambda i: (i, 0, 0)),
            pl.BlockSpec((1, 1, H), lambda i: (i, 0, 0)),
            pl.BlockSpec((1, H), lambda i: (0, 0)),
            pl.BlockSpec((1, 1, 1), lambda i: (i, 0, 0)),
            pl.BlockSpec((1, 1), lambda i: (0, 0)),
        ],
        out_specs=(
            pl.BlockSpec((1, S, H), lambda i: (i, 0, 0)),
            pl.BlockSpec((1, 1, H), lambda i: (i, 0, 0)),
            pl.BlockSpec((1, 1, T), lambda i: (i, 0, 0)),
        ),
        out_shape=out_shapes,
    )(bert_representation, mtl3, hai3, W, smf, snz)

    return (bert_out, mtl_out.reshape(B, H), probs_out.reshape(B, T))


# speedup vs baseline: 12.0562x; 1.0958x over previous
import jax
import jax.numpy as jnp
from jax.experimental import pallas as pl

B = 16
T = 11
S = 512
H = 1024
BB = 2


def _fused_step(bert_ref, mtl_ref, hai_ref, w_ref, smf_ref, snz_ref,
                bert_out_ref, mtl_out_ref, probs_out_ref):
    l = jnp.sum(hai_ref[:, 0, :] * w_ref[...], axis=1, keepdims=True)
    snz = snz_ref[0, 0]
    t = jax.lax.broadcasted_iota(jnp.int32, (1, T), 1).astype(jnp.float32)
    logits = jnp.where(t == float(T - 1), l, 0.0) + snz
    mask = (t >= (float(T) - smf_ref[:, 0, :])).astype(jnp.float32)
    e = jnp.exp(logits) * mask
    probs = e / jnp.sum(e, axis=1, keepdims=True)
    probs_out_ref[...] = probs.reshape(BB, 1, T)
    p = probs[:, T - 1].reshape(BB, 1, 1)
    mtl_out_ref[...] = mtl_ref[...] * p
    bert_out_ref[...] = bert_ref[...] * p


def kernel(bert_representation, mtl_input, slice_mask, slice_num,
           history_attention_input, W):
    smf = slice_mask.astype(jnp.float32).reshape(B, 1, 1)
    snz = (jnp.asarray(slice_num) - B).astype(jnp.float32).reshape(1, 1)
    hai3 = history_attention_input.reshape(B, 1, H)
    mtl3 = mtl_input.reshape(B, 1, H)

    out_shapes = (
        jax.ShapeDtypeStruct((B, S, H), jnp.float32),
        jax.ShapeDtypeStruct((B, 1, H), jnp.float32),
        jax.ShapeDtypeStruct((B, 1, T), jnp.float32),
    )
    bert_out, mtl_out, probs_out = pl.pallas_call(
        _fused_step,
        grid=(B // BB,),
        in_specs=[
            pl.BlockSpec((BB, S, H), lambda i: (i, 0, 0)),
            pl.BlockSpec((BB, 1, H), lambda i: (i, 0, 0)),
            pl.BlockSpec((BB, 1, H), lambda i: (i, 0, 0)),
            pl.BlockSpec((1, H), lambda i: (0, 0)),
            pl.BlockSpec((BB, 1, 1), lambda i: (i, 0, 0)),
            pl.BlockSpec((1, 1), lambda i: (0, 0)),
        ],
        out_specs=(
            pl.BlockSpec((BB, S, H), lambda i: (i, 0, 0)),
            pl.BlockSpec((BB, 1, H), lambda i: (i, 0, 0)),
            pl.BlockSpec((BB, 1, T), lambda i: (i, 0, 0)),
        ),
        out_shape=out_shapes,
    )(bert_representation, mtl3, hai3, W, smf, snz)

    return (bert_out, mtl_out.reshape(B, H), probs_out.reshape(B, T))


# speedup vs baseline: 12.4309x; 1.0311x over previous
import jax
import jax.numpy as jnp
from jax.experimental import pallas as pl

B = 16
T = 11
S = 512
H = 1024
BB = 4


def _fused_step(bert_ref, mtl_ref, hai_ref, w_ref, smf_ref, snz_ref,
                bert_out_ref, mtl_out_ref, probs_out_ref):
    l = jnp.sum(hai_ref[:, 0, :] * w_ref[...], axis=1, keepdims=True)
    snz = snz_ref[0, 0]
    t = jax.lax.broadcasted_iota(jnp.int32, (1, T), 1).astype(jnp.float32)
    logits = jnp.where(t == float(T - 1), l, 0.0) + snz
    mask = (t >= (float(T) - smf_ref[:, 0, :])).astype(jnp.float32)
    e = jnp.exp(logits) * mask
    probs = e / jnp.sum(e, axis=1, keepdims=True)
    probs_out_ref[...] = probs.reshape(BB, 1, T)
    p = probs[:, T - 1].reshape(BB, 1, 1)
    mtl_out_ref[...] = mtl_ref[...] * p
    bert_out_ref[...] = bert_ref[...] * p


def kernel(bert_representation, mtl_input, slice_mask, slice_num,
           history_attention_input, W):
    smf = slice_mask.astype(jnp.float32).reshape(B, 1, 1)
    snz = (jnp.asarray(slice_num) - B).astype(jnp.float32).reshape(1, 1)
    hai3 = history_attention_input.reshape(B, 1, H)
    mtl3 = mtl_input.reshape(B, 1, H)

    out_shapes = (
        jax.ShapeDtypeStruct((B, S, H), jnp.float32),
        jax.ShapeDtypeStruct((B, 1, H), jnp.float32),
        jax.ShapeDtypeStruct((B, 1, T), jnp.float32),
    )
    bert_out, mtl_out, probs_out = pl.pallas_call(
        _fused_step,
        grid=(B // BB,),
        in_specs=[
            pl.BlockSpec((BB, S, H), lambda i: (i, 0, 0)),
            pl.BlockSpec((BB, 1, H), lambda i: (i, 0, 0)),
            pl.BlockSpec((BB, 1, H), lambda i: (i, 0, 0)),
            pl.BlockSpec((1, H), lambda i: (0, 0)),
            pl.BlockSpec((BB, 1, 1), lambda i: (i, 0, 0)),
            pl.BlockSpec((1, 1), lambda i: (0, 0)),
        ],
        out_specs=(
            pl.BlockSpec((BB, S, H), lambda i: (i, 0, 0)),
            pl.BlockSpec((BB, 1, H), lambda i: (i, 0, 0)),
            pl.BlockSpec((BB, 1, T), lambda i: (i, 0, 0)),
        ),
        out_shape=out_shapes,
    )(bert_representation, mtl3, hai3, W, smf, snz)

    return (bert_out, mtl_out.reshape(B, H), probs_out.reshape(B, T))
